# Initial kernel scaffold; baseline (speedup 1.0000x reference)
import jax
import jax.numpy as jnp
from jax.experimental import pallas as pl


def kernel(x, W1, g1, b1, W2, g2, b2, W3, g3, b3, W4, g4, b4):
    raise NotImplementedError("write your pallas kernel here")


# speedup vs baseline: 5.1227x; 5.1227x over previous
import functools

import jax
import jax.numpy as jnp
from jax import lax
from jax.experimental import pallas as pl
from jax.experimental.pallas import tpu as pltpu
from jax.experimental.pallas import tpu_sc as plsc

KNN = 20
NEG = float("-inf")
EPS = 1e-5

NC, NS = 2, 16
NW = NC * NS
HALF = 128


def _knn_body(N, RB, x_ref, xa_ref, idx_ref):
    b = pl.program_id(0)
    xr = x_ref[0]
    xa = xa_ref[0]
    g = lax.dot_general(xr, xa, (((1,), (1,)), ((), ())))
    xx_r = jnp.sum(xr * xr, axis=1, keepdims=True)
    xx_a = jnp.sum(xa * xa, axis=1)[None, :]
    d = 2.0 * g - xx_r - xx_a
    iota = lax.broadcasted_iota(jnp.int32, (RB, N), 1)
    rows = []
    for _ in range(KNN):
        m = jnp.max(d, axis=1, keepdims=True)
        am = jnp.min(jnp.where(d == m, iota, N), axis=1)
        rows.append(am)
        d = jnp.where(iota == am[:, None], NEG, d)
    idx_ref[0] = jnp.stack(rows, axis=0) + b * N


@functools.lru_cache(maxsize=None)
def _make_knn(B, N, C, RB):
    return pl.pallas_call(
        functools.partial(_knn_body, N, RB),
        grid=(B, N // RB),
        in_specs=[
            pl.BlockSpec((1, RB, C), lambda b, r: (b, r, 0)),
            pl.BlockSpec((1, N, C), lambda b, r: (b, 0, 0)),
        ],
        out_specs=pl.BlockSpec((1, KNN, RB), lambda b, r: (b, 0, r)),
        out_shape=jax.ShapeDtypeStruct((B, KNN, N), jnp.int32),
    )


@functools.lru_cache(maxsize=None)
def _make_gather(BN, N, C):
    P = BN // NW
    SPB = N // P
    mesh = plsc.VectorSubcoreMesh(core_axis_name="c", subcore_axis_name="s",
                                  num_cores=NC, num_subcores=NS)

    @functools.partial(
        pl.kernel,
        out_type=jax.ShapeDtypeStruct((KNN * BN, C), jnp.float32),
        mesh=mesh,
        compiler_params=pltpu.CompilerParams(use_tc_tiling_on_sc=False),
        scratch_types=[
            pltpu.VMEM((P,), jnp.int32),
            pltpu.VMEM((2, HALF, C), jnp.float32),
            pltpu.SemaphoreType.DMA,
            pltpu.SemaphoreType.DMA,
            pltpu.SemaphoreType.DMA,
            pltpu.SemaphoreType.DMA,
        ],
    )
    def gather(x_hbm, idxt_hbm, g_hbm, idx_v, buf_v, gs0, gs1, ss0, ss1):
        wid = lax.axis_index("s") * NC + lax.axis_index("c")
        b = lax.div(wid, SPB)
        nbase = lax.rem(wid, SPB) * P
        pbase = wid * P
        gsems = (gs0, gs1)
        ssems = (ss0, ss1)

        def body(j, _):
            pltpu.sync_copy(
                idxt_hbm.at[pl.ds((b * KNN + j) * N + nbase, P)], idx_v)
            for h in (0, 1):
                pltpu.make_async_copy(
                    x_hbm.at[idx_v.at[pl.ds(h * HALF, HALF)]],
                    buf_v.at[h], gsems[h]).start()
            for h in (0, 1):
                pltpu.make_async_copy(
                    x_hbm.at[idx_v.at[pl.ds(h * HALF, HALF)]],
                    buf_v.at[h], gsems[h]).wait()
                row0 = j * BN + pbase + h * HALF
                pltpu.make_async_copy(
                    buf_v.at[h], g_hbm.at[pl.ds(row0, HALF)], ssems[h]).start()
            for h in (0, 1):
                row0 = j * BN + pbase + h * HALF
                pltpu.make_async_copy(
                    buf_v.at[h], g_hbm.at[pl.ds(row0, HALF)], ssems[h]).wait()
            return 0

        lax.fori_loop(0, KNN, body, 0)

    return gather


def _conv_reduce_body(x_ref, g_ref, wd_ref, wx_ref, m_ref, sums_ref):
    a = x_ref[0]
    acen = lax.dot_general(a, wx_ref[...], (((1,), (1,)), ((), ())))
    m = jnp.full(acen.shape, NEG, jnp.float32)
    s = jnp.zeros_like(acen)
    q = jnp.zeros_like(acen)
    for j in range(KNN):
        dif = g_ref[j, 0] - a
        y = lax.dot_general(dif, wd_ref[...], (((1,), (1,)), ((), ()))) + acen
        m = jnp.maximum(m, y)
        s = s + y
        q = q + y * y
    m_ref[0] = m
    zero = jnp.zeros((s.shape[1],), jnp.float32)
    part = jnp.stack([jnp.sum(s, axis=0), jnp.sum(q, axis=0),
                      zero, zero, zero, zero, zero, zero], axis=0)

    @pl.when(jnp.logical_and(pl.program_id(0) == 0, pl.program_id(1) == 0))
    def _():
        sums_ref[...] = jnp.zeros_like(sums_ref)

    sums_ref[...] += part


@functools.lru_cache(maxsize=None)
def _make_conv_reduce(B, N, C, O, RB):
    return pl.pallas_call(
        _conv_reduce_body,
        grid=(B, N // RB),
        in_specs=[
            pl.BlockSpec((1, RB, C), lambda b, r: (b, r, 0)),
            pl.BlockSpec((KNN, 1, RB, C), lambda b, r: (0, b, r, 0)),
            pl.BlockSpec((O, C), lambda b, r: (0, 0)),
            pl.BlockSpec((O, C), lambda b, r: (0, 0)),
        ],
        out_specs=[
            pl.BlockSpec((1, RB, O), lambda b, r: (b, r, 0)),
            pl.BlockSpec((8, O), lambda b, r: (0, 0)),
        ],
        out_shape=[
            jax.ShapeDtypeStruct((B, N, O), jnp.float32),
            jax.ShapeDtypeStruct((8, O), jnp.float32),
        ],
    )


def _norm_body(T, m_ref, sums_ref, g_ref, bta_ref, out_ref):
    sums = sums_ref[...]
    mean = sums[0] / T
    var = sums[1] / T - mean * mean
    inv = lax.rsqrt(var + EPS)
    y = (m_ref[0] - mean[None, :]) * inv[None, :] * g_ref[...] + bta_ref[...]
    out_ref[0] = jnp.where(y > 0, y, 0.2 * y)


@functools.lru_cache(maxsize=None)
def _make_norm(B, N, O):
    return pl.pallas_call(
        functools.partial(_norm_body, float(B * N * KNN)),
        grid=(B,),
        in_specs=[
            pl.BlockSpec((1, N, O), lambda b: (b, 0, 0)),
            pl.BlockSpec((8, O), lambda b: (0, 0)),
            pl.BlockSpec((1, O), lambda b: (0, 0)),
            pl.BlockSpec((1, O), lambda b: (0, 0)),
        ],
        out_specs=pl.BlockSpec((1, N, O), lambda b: (b, 0, 0)),
        out_shape=jax.ShapeDtypeStruct((B, N, O), jnp.float32),
    )


def _edge_conv(xp, W, gam, bet, RB=256):
    B, N, C = xp.shape
    BN = B * N
    O, twoc = W.shape
    craw = twoc // 2
    wd = jnp.zeros((O, C), jnp.float32).at[:, :craw].set(W[:, :craw])
    wx = jnp.zeros((O, C), jnp.float32).at[:, :craw].set(W[:, craw:])
    idxt = _make_knn(B, N, C, RB)(xp, xp)
    g = _make_gather(BN, N, C)(xp.reshape(BN, C), idxt.reshape(B * KNN * N))
    m, sums = _make_conv_reduce(B, N, C, O, RB)(
        xp, g.reshape(KNN, B, N, C), wd, wx)
    return _make_norm(B, N, O)(m, sums, gam.reshape(1, O), bet.reshape(1, O))


def kernel(x, W1, g1, b1, W2, g2, b2, W3, g3, b3, W4, g4, b4):
    B, N, C0 = x.shape
    xp = jnp.pad(x, ((0, 0), (0, 0), (0, 16 - C0)))
    outs = []
    for W, gam, bet in ((W1, g1, b1), (W2, g2, b2), (W3, g3, b3), (W4, g4, b4)):
        xp = _edge_conv(xp, W, gam, bet)
        outs.append(xp)
    return jnp.concatenate(outs, axis=-1)


# speedup vs baseline: 5.3802x; 1.0503x over previous
import functools

import jax
import jax.numpy as jnp
from jax import lax
from jax.experimental import pallas as pl
from jax.experimental.pallas import tpu as pltpu
from jax.experimental.pallas import tpu_sc as plsc

KNN = 20
NEG = float("-inf")
EPS = 1e-5

NC, NS = 2, 16
NW = NC * NS
HALF = 128


def _knn_body(N, x_ref, idx_ref):
    b = pl.program_id(0)
    xr = x_ref[0]
    g = lax.dot_general(xr, xr, (((1,), (1,)), ((), ())))
    xx = jnp.sum(xr * xr, axis=1, keepdims=True)
    d = 2.0 * g - xx - xx.reshape(1, N)
    iota = lax.broadcasted_iota(jnp.int32, (N, N), 1)
    rows = []
    for _ in range(KNN):
        m = jnp.max(d, axis=1, keepdims=True)
        cand = jnp.where(d == m, iota, N)
        am = jnp.min(cand, axis=1)
        rows.append(am)
        d = jnp.where(cand == am[:, None], NEG, d)
    idx_ref[0] = jnp.stack(rows, axis=0) + b * N


@functools.lru_cache(maxsize=None)
def _make_knn(B, N, C, RB=None):
    return pl.pallas_call(
        functools.partial(_knn_body, N),
        grid=(B,),
        in_specs=[pl.BlockSpec((1, N, C), lambda b: (b, 0, 0))],
        out_specs=pl.BlockSpec((1, KNN, N), lambda b: (b, 0, 0)),
        out_shape=jax.ShapeDtypeStruct((B, KNN, N), jnp.int32),
    )


@functools.lru_cache(maxsize=None)
def _make_gather(BN, N, C):
    P = BN // NW
    SPB = N // P
    mesh = plsc.VectorSubcoreMesh(core_axis_name="c", subcore_axis_name="s",
                                  num_cores=NC, num_subcores=NS)

    @functools.partial(
        pl.kernel,
        out_type=jax.ShapeDtypeStruct((KNN * BN, C), jnp.float32),
        mesh=mesh,
        compiler_params=pltpu.CompilerParams(use_tc_tiling_on_sc=False),
        scratch_types=[
            pltpu.VMEM((P,), jnp.int32),
            pltpu.VMEM((2, HALF, C), jnp.float32),
            pltpu.SemaphoreType.DMA,
            pltpu.SemaphoreType.DMA,
            pltpu.SemaphoreType.DMA,
            pltpu.SemaphoreType.DMA,
        ],
    )
    def gather(x_hbm, idxt_hbm, g_hbm, idx_v, buf_v, gs0, gs1, ss0, ss1):
        wid = lax.axis_index("s") * NC + lax.axis_index("c")
        b = lax.div(wid, SPB)
        nbase = lax.rem(wid, SPB) * P
        pbase = wid * P
        gsems = (gs0, gs1)
        ssems = (ss0, ss1)

        def body(j, _):
            pltpu.sync_copy(
                idxt_hbm.at[pl.ds((b * KNN + j) * N + nbase, P)], idx_v)
            for h in (0, 1):
                pltpu.make_async_copy(
                    x_hbm.at[idx_v.at[pl.ds(h * HALF, HALF)]],
                    buf_v.at[h], gsems[h]).start()
            for h in (0, 1):
                pltpu.make_async_copy(
                    x_hbm.at[idx_v.at[pl.ds(h * HALF, HALF)]],
                    buf_v.at[h], gsems[h]).wait()
                row0 = j * BN + pbase + h * HALF
                pltpu.make_async_copy(
                    buf_v.at[h], g_hbm.at[pl.ds(row0, HALF)], ssems[h]).start()
            for h in (0, 1):
                row0 = j * BN + pbase + h * HALF
                pltpu.make_async_copy(
                    buf_v.at[h], g_hbm.at[pl.ds(row0, HALF)], ssems[h]).wait()
            return 0

        lax.fori_loop(0, KNN, body, 0)

    return gather


def _conv_reduce_body(x_ref, g_ref, wd_ref, wx_ref, m_ref, sums_ref):
    a = x_ref[0]
    acen = lax.dot_general(a, wx_ref[...], (((1,), (1,)), ((), ())))
    m = jnp.full(acen.shape, NEG, jnp.float32)
    s = jnp.zeros_like(acen)
    q = jnp.zeros_like(acen)
    for j in range(KNN):
        dif = g_ref[j, 0] - a
        y = lax.dot_general(dif, wd_ref[...], (((1,), (1,)), ((), ()))) + acen
        m = jnp.maximum(m, y)
        s = s + y
        q = q + y * y
    m_ref[0] = m
    zero = jnp.zeros((s.shape[1],), jnp.float32)
    part = jnp.stack([jnp.sum(s, axis=0), jnp.sum(q, axis=0),
                      zero, zero, zero, zero, zero, zero], axis=0)

    @pl.when(jnp.logical_and(pl.program_id(0) == 0, pl.program_id(1) == 0))
    def _():
        sums_ref[...] = jnp.zeros_like(sums_ref)

    sums_ref[...] += part


@functools.lru_cache(maxsize=None)
def _make_conv_reduce(B, N, C, O, RB):
    return pl.pallas_call(
        _conv_reduce_body,
        grid=(B, N // RB),
        in_specs=[
            pl.BlockSpec((1, RB, C), lambda b, r: (b, r, 0)),
            pl.BlockSpec((KNN, 1, RB, C), lambda b, r: (0, b, r, 0)),
            pl.BlockSpec((O, C), lambda b, r: (0, 0)),
            pl.BlockSpec((O, C), lambda b, r: (0, 0)),
        ],
        out_specs=[
            pl.BlockSpec((1, RB, O), lambda b, r: (b, r, 0)),
            pl.BlockSpec((8, O), lambda b, r: (0, 0)),
        ],
        out_shape=[
            jax.ShapeDtypeStruct((B, N, O), jnp.float32),
            jax.ShapeDtypeStruct((8, O), jnp.float32),
        ],
    )


def _norm_body(T, m_ref, sums_ref, g_ref, bta_ref, out_ref):
    sums = sums_ref[...]
    mean = sums[0] / T
    var = sums[1] / T - mean * mean
    inv = lax.rsqrt(var + EPS)
    y = (m_ref[0] - mean[None, :]) * inv[None, :] * g_ref[...] + bta_ref[...]
    out_ref[0] = jnp.where(y > 0, y, 0.2 * y)


@functools.lru_cache(maxsize=None)
def _make_norm(B, N, O):
    return pl.pallas_call(
        functools.partial(_norm_body, float(B * N * KNN)),
        grid=(B,),
        in_specs=[
            pl.BlockSpec((1, N, O), lambda b: (b, 0, 0)),
            pl.BlockSpec((8, O), lambda b: (0, 0)),
            pl.BlockSpec((1, O), lambda b: (0, 0)),
            pl.BlockSpec((1, O), lambda b: (0, 0)),
        ],
        out_specs=pl.BlockSpec((1, N, O), lambda b: (b, 0, 0)),
        out_shape=jax.ShapeDtypeStruct((B, N, O), jnp.float32),
    )


def _edge_conv(xp, W, gam, bet, RB=256):
    B, N, C = xp.shape
    BN = B * N
    O, twoc = W.shape
    craw = twoc // 2
    wd = jnp.zeros((O, C), jnp.float32).at[:, :craw].set(W[:, :craw])
    wx = jnp.zeros((O, C), jnp.float32).at[:, :craw].set(W[:, craw:])
    idxt = _make_knn(B, N, C)(xp)
    g = _make_gather(BN, N, C)(xp.reshape(BN, C), idxt.reshape(B * KNN * N))
    m, sums = _make_conv_reduce(B, N, C, O, RB)(
        xp, g.reshape(KNN, B, N, C), wd, wx)
    return _make_norm(B, N, O)(m, sums, gam.reshape(1, O), bet.reshape(1, O))


def kernel(x, W1, g1, b1, W2, g2, b2, W3, g3, b3, W4, g4, b4):
    B, N, C0 = x.shape
    xp = jnp.pad(x, ((0, 0), (0, 0), (0, 16 - C0)))
    outs = []
    for W, gam, bet in ((W1, g1, b1), (W2, g2, b2), (W3, g3, b3), (W4, g4, b4)):
        xp = _edge_conv(xp, W, gam, bet)
        outs.append(xp)
    return jnp.concatenate(outs, axis=-1)


# speedup vs baseline: 5.5642x; 1.0342x over previous
import functools

import jax
import jax.numpy as jnp
from jax import lax
from jax.experimental import pallas as pl
from jax.experimental.pallas import tpu as pltpu
from jax.experimental.pallas import tpu_sc as plsc

KNN = 20
NEG = float("-inf")
EPS = 1e-5

NC, NS = 2, 16
NW = NC * NS
HALF = 128


def _knn_body(N, x_ref, idx_ref):
    b = pl.program_id(0)
    xr = x_ref[0]
    g = lax.dot_general(xr, xr, (((1,), (1,)), ((), ())))
    xx = jnp.sum(xr * xr, axis=1, keepdims=True)
    d = 2.0 * g - xx - xx.reshape(1, N)
    iota = lax.broadcasted_iota(jnp.int32, (N, N), 1)
    rows = []
    for _ in range(KNN):
        m = jnp.max(d, axis=1, keepdims=True)
        cand = jnp.where(d == m, iota, N)
        am = jnp.min(cand, axis=1)
        rows.append(am)
        d = jnp.where(cand == am[:, None], NEG, d)
    idx_ref[0] = jnp.stack(rows, axis=0) + b * N


@functools.lru_cache(maxsize=None)
def _make_knn(B, N, C, RB=None):
    return pl.pallas_call(
        functools.partial(_knn_body, N),
        grid=(B,),
        in_specs=[pl.BlockSpec((1, N, C), lambda b: (b, 0, 0))],
        out_specs=pl.BlockSpec((1, KNN, N), lambda b: (b, 0, 0)),
        out_shape=jax.ShapeDtypeStruct((B, KNN, N), jnp.int32),
    )


@functools.lru_cache(maxsize=None)
def _make_gather(BN, N, C):
    P = BN // NW
    SPB = N // P
    mesh = plsc.VectorSubcoreMesh(core_axis_name="c", subcore_axis_name="s",
                                  num_cores=NC, num_subcores=NS)

    @functools.partial(
        pl.kernel,
        out_type=jax.ShapeDtypeStruct((KNN * BN, C), jnp.float32),
        mesh=mesh,
        compiler_params=pltpu.CompilerParams(use_tc_tiling_on_sc=False),
        scratch_types=[
            pltpu.VMEM((KNN, P), jnp.int32),
            pltpu.VMEM((4, HALF, C), jnp.float32),
            pltpu.SemaphoreType.DMA,
            pltpu.SemaphoreType.DMA,
            pltpu.SemaphoreType.DMA,
            pltpu.SemaphoreType.DMA,
            pltpu.SemaphoreType.DMA,
            pltpu.SemaphoreType.DMA,
            pltpu.SemaphoreType.DMA,
            pltpu.SemaphoreType.DMA,
        ],
    )
    def gather(x_hbm, idxt_hbm, g_hbm, idx_v, buf_v, *sems):
        wid = lax.axis_index("s") * NC + lax.axis_index("c")
        b = lax.div(wid, SPB)
        nbase = lax.rem(wid, SPB) * P
        pbase = wid * P
        gsems = sems[:4]
        ssems = sems[4:]
        pltpu.sync_copy(
            idxt_hbm.at[pl.ds(b * KNN, KNN), pl.ds(nbase, P)], idx_v)

        def gath(j, h):
            buf = (j % 2) * 2 + h
            return pltpu.make_async_copy(
                x_hbm.at[idx_v.at[j, pl.ds(h * HALF, HALF)]],
                buf_v.at[buf], gsems[buf])

        def stor(j, h):
            buf = (j % 2) * 2 + h
            return pltpu.make_async_copy(
                buf_v.at[buf],
                g_hbm.at[pl.ds(j * BN + pbase + h * HALF, HALF)], ssems[buf])

        for h in (0, 1):
            gath(0, h).start()
        for j in range(KNN):
            for h in (0, 1):
                gath(j, h).wait()
                stor(j, h).start()
            if j + 1 < KNN:
                for h in (0, 1):
                    if j >= 1:
                        stor(j - 1, h).wait()
                    gath(j + 1, h).start()
        for h in (0, 1):
            stor(KNN - 2, h).wait()
            stor(KNN - 1, h).wait()

    return gather


def _conv_reduce_body(x_ref, g_ref, wd_ref, wx_ref, m_ref, sums_ref):
    a = x_ref[0]
    acen = lax.dot_general(a, wx_ref[...], (((1,), (1,)), ((), ())))
    m = jnp.full(acen.shape, NEG, jnp.float32)
    s = jnp.zeros_like(acen)
    q = jnp.zeros_like(acen)
    for j in range(KNN):
        dif = g_ref[j, 0] - a
        y = lax.dot_general(dif, wd_ref[...], (((1,), (1,)), ((), ()))) + acen
        m = jnp.maximum(m, y)
        s = s + y
        q = q + y * y
    m_ref[0] = m
    zero = jnp.zeros((s.shape[1],), jnp.float32)
    part = jnp.stack([jnp.sum(s, axis=0), jnp.sum(q, axis=0),
                      zero, zero, zero, zero, zero, zero], axis=0)

    @pl.when(jnp.logical_and(pl.program_id(0) == 0, pl.program_id(1) == 0))
    def _():
        sums_ref[...] = jnp.zeros_like(sums_ref)

    sums_ref[...] += part


@functools.lru_cache(maxsize=None)
def _make_conv_reduce(B, N, C, O, RB):
    return pl.pallas_call(
        _conv_reduce_body,
        grid=(B, N // RB),
        in_specs=[
            pl.BlockSpec((1, RB, C), lambda b, r: (b, r, 0)),
            pl.BlockSpec((KNN, 1, RB, C), lambda b, r: (0, b, r, 0)),
            pl.BlockSpec((O, C), lambda b, r: (0, 0)),
            pl.BlockSpec((O, C), lambda b, r: (0, 0)),
        ],
        out_specs=[
            pl.BlockSpec((1, RB, O), lambda b, r: (b, r, 0)),
            pl.BlockSpec((8, O), lambda b, r: (0, 0)),
        ],
        out_shape=[
            jax.ShapeDtypeStruct((B, N, O), jnp.float32),
            jax.ShapeDtypeStruct((8, O), jnp.float32),
        ],
    )


def _norm_body(T, m_ref, sums_ref, g_ref, bta_ref, out_ref):
    sums = sums_ref[...]
    mean = sums[0] / T
    var = sums[1] / T - mean * mean
    inv = lax.rsqrt(var + EPS)
    y = (m_ref[0] - mean[None, :]) * inv[None, :] * g_ref[...] + bta_ref[...]
    out_ref[0] = jnp.where(y > 0, y, 0.2 * y)


@functools.lru_cache(maxsize=None)
def _make_norm(B, N, O):
    return pl.pallas_call(
        functools.partial(_norm_body, float(B * N * KNN)),
        grid=(B,),
        in_specs=[
            pl.BlockSpec((1, N, O), lambda b: (b, 0, 0)),
            pl.BlockSpec((8, O), lambda b: (0, 0)),
            pl.BlockSpec((1, O), lambda b: (0, 0)),
            pl.BlockSpec((1, O), lambda b: (0, 0)),
        ],
        out_specs=pl.BlockSpec((1, N, O), lambda b: (b, 0, 0)),
        out_shape=jax.ShapeDtypeStruct((B, N, O), jnp.float32),
    )


def _edge_conv(xp, W, gam, bet, RB=256):
    B, N, C = xp.shape
    BN = B * N
    O, twoc = W.shape
    craw = twoc // 2
    wd = jnp.zeros((O, C), jnp.float32).at[:, :craw].set(W[:, :craw])
    wx = jnp.zeros((O, C), jnp.float32).at[:, :craw].set(W[:, craw:])
    idxt = _make_knn(B, N, C)(xp)
    g = _make_gather(BN, N, C)(xp.reshape(BN, C), idxt.reshape(B * KNN, N))
    m, sums = _make_conv_reduce(B, N, C, O, RB)(
        xp, g.reshape(KNN, B, N, C), wd, wx)
    return _make_norm(B, N, O)(m, sums, gam.reshape(1, O), bet.reshape(1, O))


def kernel(x, W1, g1, b1, W2, g2, b2, W3, g3, b3, W4, g4, b4):
    B, N, C0 = x.shape
    xp = jnp.pad(x, ((0, 0), (0, 0), (0, 16 - C0)))
    outs = []
    for W, gam, bet in ((W1, g1, b1), (W2, g2, b2), (W3, g3, b3), (W4, g4, b4)):
        xp = _edge_conv(xp, W, gam, bet)
        outs.append(xp)
    return jnp.concatenate(outs, axis=-1)


# speedup vs baseline: 6.5384x; 1.1751x over previous
import functools

import jax
import jax.numpy as jnp
from jax import lax
from jax.experimental import pallas as pl
from jax.experimental.pallas import tpu as pltpu
from jax.experimental.pallas import tpu_sc as plsc

KNN = 20
NEG = float("-inf")
EPS = 1e-5

NC, NS = 2, 16
NW = NC * NS
HALF = 128


def _knn_body(N, x_ref, idx_ref):
    b = pl.program_id(0)
    xr = x_ref[0]
    g = lax.dot_general(xr, xr, (((1,), (1,)), ((), ())))
    xx = jnp.sum(xr * xr, axis=1, keepdims=True)
    d = 2.0 * g - xx - xx.reshape(1, N)
    iota = lax.broadcasted_iota(jnp.int32, (N, N), 1).astype(jnp.float32)
    rows = []
    for _ in range(KNN):
        m = jnp.max(d, axis=1, keepdims=True)
        msk = d == m
        am = jnp.min(jnp.where(msk, iota, float(N)), axis=1)
        rows.append(am)
        d = jnp.where(msk, NEG, d)
    idx = jnp.stack(rows, axis=0).astype(jnp.int32)
    idx_ref[0] = idx + b * N


@functools.lru_cache(maxsize=None)
def _make_knn(B, N, C, RB=None):
    return pl.pallas_call(
        functools.partial(_knn_body, N),
        grid=(B,),
        in_specs=[pl.BlockSpec((1, N, C), lambda b: (b, 0, 0))],
        out_specs=pl.BlockSpec((1, KNN, N), lambda b: (b, 0, 0)),
        out_shape=jax.ShapeDtypeStruct((B, KNN, N), jnp.int32),
    )


@functools.lru_cache(maxsize=None)
def _make_gather(BN, N, C):
    P = BN // NW
    SPB = N // P
    mesh = plsc.VectorSubcoreMesh(core_axis_name="c", subcore_axis_name="s",
                                  num_cores=NC, num_subcores=NS)

    @functools.partial(
        pl.kernel,
        out_type=jax.ShapeDtypeStruct((KNN * BN, C), jnp.float32),
        mesh=mesh,
        compiler_params=pltpu.CompilerParams(use_tc_tiling_on_sc=False),
        scratch_types=[
            pltpu.VMEM((KNN, P), jnp.int32),
            pltpu.VMEM((4, HALF, C), jnp.float32),
            pltpu.SemaphoreType.DMA,
            pltpu.SemaphoreType.DMA,
            pltpu.SemaphoreType.DMA,
            pltpu.SemaphoreType.DMA,
            pltpu.SemaphoreType.DMA,
            pltpu.SemaphoreType.DMA,
            pltpu.SemaphoreType.DMA,
            pltpu.SemaphoreType.DMA,
        ],
    )
    def gather(x_hbm, idxt_hbm, g_hbm, idx_v, buf_v, *sems):
        wid = lax.axis_index("s") * NC + lax.axis_index("c")
        b = lax.div(wid, SPB)
        nbase = lax.rem(wid, SPB) * P
        pbase = wid * P
        gsems = sems[:4]
        ssems = sems[4:]
        pltpu.sync_copy(
            idxt_hbm.at[pl.ds(b * KNN, KNN), pl.ds(nbase, P)], idx_v)

        def gath(j, h):
            buf = (j % 2) * 2 + h
            return pltpu.make_async_copy(
                x_hbm.at[idx_v.at[j, pl.ds(h * HALF, HALF)]],
                buf_v.at[buf], gsems[buf])

        def stor(j, h):
            buf = (j % 2) * 2 + h
            return pltpu.make_async_copy(
                buf_v.at[buf],
                g_hbm.at[pl.ds(j * BN + pbase + h * HALF, HALF)], ssems[buf])

        for h in (0, 1):
            gath(0, h).start()
        for j in range(KNN):
            for h in (0, 1):
                gath(j, h).wait()
                stor(j, h).start()
            if j + 1 < KNN:
                for h in (0, 1):
                    if j >= 1:
                        stor(j - 1, h).wait()
                    gath(j + 1, h).start()
        for h in (0, 1):
            stor(KNN - 2, h).wait()
            stor(KNN - 1, h).wait()

    return gather


def _conv_reduce_body(x_ref, g_ref, wd_ref, wx_ref, m_ref, sums_ref):
    a = x_ref[0]
    acen = lax.dot_general(a, wx_ref[...], (((1,), (1,)), ((), ())))
    m = jnp.full(acen.shape, NEG, jnp.float32)
    s = jnp.zeros_like(acen)
    q = jnp.zeros_like(acen)
    for j in range(KNN):
        dif = g_ref[j, 0] - a
        y = lax.dot_general(dif, wd_ref[...], (((1,), (1,)), ((), ()))) + acen
        m = jnp.maximum(m, y)
        s = s + y
        q = q + y * y
    m_ref[0] = m
    zero = jnp.zeros((s.shape[1],), jnp.float32)
    part = jnp.stack([jnp.sum(s, axis=0), jnp.sum(q, axis=0),
                      zero, zero, zero, zero, zero, zero], axis=0)

    @pl.when(jnp.logical_and(pl.program_id(0) == 0, pl.program_id(1) == 0))
    def _():
        sums_ref[...] = jnp.zeros_like(sums_ref)

    sums_ref[...] += part


@functools.lru_cache(maxsize=None)
def _make_conv_reduce(B, N, C, O, RB):
    return pl.pallas_call(
        _conv_reduce_body,
        grid=(B, N // RB),
        in_specs=[
            pl.BlockSpec((1, RB, C), lambda b, r: (b, r, 0)),
            pl.BlockSpec((KNN, 1, RB, C), lambda b, r: (0, b, r, 0)),
            pl.BlockSpec((O, C), lambda b, r: (0, 0)),
            pl.BlockSpec((O, C), lambda b, r: (0, 0)),
        ],
        out_specs=[
            pl.BlockSpec((1, RB, O), lambda b, r: (b, r, 0)),
            pl.BlockSpec((8, O), lambda b, r: (0, 0)),
        ],
        out_shape=[
            jax.ShapeDtypeStruct((B, N, O), jnp.float32),
            jax.ShapeDtypeStruct((8, O), jnp.float32),
        ],
    )


def _norm_body(T, m_ref, sums_ref, g_ref, bta_ref, out_ref):
    sums = sums_ref[...]
    mean = sums[0] / T
    var = sums[1] / T - mean * mean
    inv = lax.rsqrt(var + EPS)
    y = (m_ref[0] - mean[None, :]) * inv[None, :] * g_ref[...] + bta_ref[...]
    out_ref[0] = jnp.where(y > 0, y, 0.2 * y)


@functools.lru_cache(maxsize=None)
def _make_norm(B, N, O):
    return pl.pallas_call(
        functools.partial(_norm_body, float(B * N * KNN)),
        grid=(B,),
        in_specs=[
            pl.BlockSpec((1, N, O), lambda b: (b, 0, 0)),
            pl.BlockSpec((8, O), lambda b: (0, 0)),
            pl.BlockSpec((1, O), lambda b: (0, 0)),
            pl.BlockSpec((1, O), lambda b: (0, 0)),
        ],
        out_specs=pl.BlockSpec((1, N, O), lambda b: (b, 0, 0)),
        out_shape=jax.ShapeDtypeStruct((B, N, O), jnp.float32),
    )


def _edge_conv(xp, W, gam, bet, RB=256):
    B, N, C = xp.shape
    BN = B * N
    O, twoc = W.shape
    craw = twoc // 2
    wd = jnp.zeros((O, C), jnp.float32).at[:, :craw].set(W[:, :craw])
    wx = jnp.zeros((O, C), jnp.float32).at[:, :craw].set(W[:, craw:])
    idxt = _make_knn(B, N, C)(xp)
    g = _make_gather(BN, N, C)(xp.reshape(BN, C), idxt.reshape(B * KNN, N))
    m, sums = _make_conv_reduce(B, N, C, O, RB)(
        xp, g.reshape(KNN, B, N, C), wd, wx)
    return _make_norm(B, N, O)(m, sums, gam.reshape(1, O), bet.reshape(1, O))


def kernel(x, W1, g1, b1, W2, g2, b2, W3, g3, b3, W4, g4, b4):
    B, N, C0 = x.shape
    xp = jnp.pad(x, ((0, 0), (0, 0), (0, 16 - C0)))
    outs = []
    for W, gam, bet in ((W1, g1, b1), (W2, g2, b2), (W3, g3, b3), (W4, g4, b4)):
        xp = _edge_conv(xp, W, gam, bet)
        outs.append(xp)
    return jnp.concatenate(outs, axis=-1)


# speedup vs baseline: 6.9643x; 1.0651x over previous
import functools

import jax
import jax.numpy as jnp
from jax import lax
from jax.experimental import pallas as pl
from jax.experimental.pallas import tpu as pltpu
from jax.experimental.pallas import tpu_sc as plsc

KNN = 20
NEG = float("-inf")
EPS = 1e-5

NC, NS = 2, 16
NW = NC * NS
HALF = 128


def _knn_body(N, x_ref, idx_ref):
    b = pl.program_id(0)
    xr = x_ref[0]
    g = lax.dot_general(xr, xr, (((1,), (1,)), ((), ())))
    xx = jnp.sum(xr * xr, axis=1, keepdims=True)
    d = 2.0 * g - xx - xx.reshape(1, N)
    iota = lax.broadcasted_iota(jnp.int32, (N, N), 1).astype(jnp.float32)
    rows = []
    for _ in range(KNN):
        m = jnp.max(d, axis=1, keepdims=True)
        msk = d == m
        am = jnp.min(jnp.where(msk, iota, float(N)), axis=1)
        rows.append(am)
        d = jnp.where(msk, NEG, d)
    idx = jnp.stack(rows, axis=0).astype(jnp.int32)
    idx_ref[0] = idx + b * N


@functools.lru_cache(maxsize=None)
def _make_knn(B, N, C, RB=None):
    return pl.pallas_call(
        functools.partial(_knn_body, N),
        grid=(B,),
        in_specs=[pl.BlockSpec((1, N, C), lambda b: (b, 0, 0))],
        out_specs=pl.BlockSpec((1, KNN, N), lambda b: (b, 0, 0)),
        out_shape=jax.ShapeDtypeStruct((B, KNN, N), jnp.int32),
    )


@functools.lru_cache(maxsize=None)
def _make_gather(BN, N, C):
    P = BN // NW
    SPB = N // P
    mesh = plsc.VectorSubcoreMesh(core_axis_name="c", subcore_axis_name="s",
                                  num_cores=NC, num_subcores=NS)

    @functools.partial(
        pl.kernel,
        out_type=jax.ShapeDtypeStruct((KNN * BN, C), jnp.float32),
        mesh=mesh,
        compiler_params=pltpu.CompilerParams(use_tc_tiling_on_sc=False),
        scratch_types=[
            pltpu.VMEM((KNN, P), jnp.int32),
            pltpu.VMEM((4, HALF, C), jnp.float32),
            pltpu.SemaphoreType.DMA,
            pltpu.SemaphoreType.DMA,
            pltpu.SemaphoreType.DMA,
            pltpu.SemaphoreType.DMA,
            pltpu.SemaphoreType.DMA,
            pltpu.SemaphoreType.DMA,
            pltpu.SemaphoreType.DMA,
            pltpu.SemaphoreType.DMA,
        ],
    )
    def gather(x_hbm, idxt_hbm, g_hbm, idx_v, buf_v, *sems):
        wid = lax.axis_index("s") * NC + lax.axis_index("c")
        b = lax.div(wid, SPB)
        nbase = lax.rem(wid, SPB) * P
        pbase = wid * P
        gsems = sems[:4]
        ssems = sems[4:]
        pltpu.sync_copy(
            idxt_hbm.at[pl.ds(b * KNN, KNN), pl.ds(nbase, P)], idx_v)

        def gath(j, h):
            buf = (j % 2) * 2 + h
            return pltpu.make_async_copy(
                x_hbm.at[idx_v.at[j, pl.ds(h * HALF, HALF)]],
                buf_v.at[buf], gsems[buf])

        def stor(j, h):
            buf = (j % 2) * 2 + h
            return pltpu.make_async_copy(
                buf_v.at[buf],
                g_hbm.at[pl.ds(j * BN + pbase + h * HALF, HALF)], ssems[buf])

        for h in (0, 1):
            gath(0, h).start()
        for j in range(KNN):
            for h in (0, 1):
                gath(j, h).wait()
                stor(j, h).start()
            if j + 1 < KNN:
                for h in (0, 1):
                    if j >= 1:
                        stor(j - 1, h).wait()
                    gath(j + 1, h).start()
        for h in (0, 1):
            stor(KNN - 2, h).wait()
            stor(KNN - 1, h).wait()

    return gather


def _conv_reduce_body(x_ref, g_ref, wd_ref, wx_ref, m_ref, sums_ref):
    a = x_ref[0]
    acen = lax.dot_general(a, wx_ref[...], (((1,), (1,)), ((), ())))
    m = jnp.full(acen.shape, NEG, jnp.float32)
    s = jnp.zeros_like(acen)
    q = jnp.zeros_like(acen)
    for j in range(KNN):
        dif = g_ref[j, 0] - a
        y = lax.dot_general(dif, wd_ref[...], (((1,), (1,)), ((), ()))) + acen
        m = jnp.maximum(m, y)
        s = s + y
        q = q + y * y
    m_ref[0] = m
    zero = jnp.zeros((s.shape[1],), jnp.float32)
    part = jnp.stack([jnp.sum(s, axis=0), jnp.sum(q, axis=0),
                      zero, zero, zero, zero, zero, zero], axis=0)

    @pl.when(jnp.logical_and(pl.program_id(0) == 0, pl.program_id(1) == 0))
    def _():
        sums_ref[...] = jnp.zeros_like(sums_ref)

    sums_ref[...] += part


@functools.lru_cache(maxsize=None)
def _make_conv_reduce(B, N, C, O, RB):
    return pl.pallas_call(
        _conv_reduce_body,
        grid=(B, N // RB),
        in_specs=[
            pl.BlockSpec((1, RB, C), lambda b, r: (b, r, 0)),
            pl.BlockSpec((KNN, 1, RB, C), lambda b, r: (0, b, r, 0)),
            pl.BlockSpec((O, C), lambda b, r: (0, 0)),
            pl.BlockSpec((O, C), lambda b, r: (0, 0)),
        ],
        out_specs=[
            pl.BlockSpec((1, RB, O), lambda b, r: (b, r, 0)),
            pl.BlockSpec((8, O), lambda b, r: (0, 0)),
        ],
        out_shape=[
            jax.ShapeDtypeStruct((B, N, O), jnp.float32),
            jax.ShapeDtypeStruct((8, O), jnp.float32),
        ],
    )


def _norm_body(T, m_ref, sums_ref, g_ref, bta_ref, out_ref):
    sums = sums_ref[...]
    mean = sums[0] / T
    var = sums[1] / T - mean * mean
    inv = lax.rsqrt(var + EPS)
    y = (m_ref[0] - mean[None, :]) * inv[None, :] * g_ref[...] + bta_ref[...]
    out_ref[0] = jnp.where(y > 0, y, 0.2 * y)


@functools.lru_cache(maxsize=None)
def _make_norm(B, N, O):
    return pl.pallas_call(
        functools.partial(_norm_body, float(B * N * KNN)),
        grid=(B,),
        in_specs=[
            pl.BlockSpec((1, N, O), lambda b: (b, 0, 0)),
            pl.BlockSpec((8, O), lambda b: (0, 0)),
            pl.BlockSpec((1, O), lambda b: (0, 0)),
            pl.BlockSpec((1, O), lambda b: (0, 0)),
        ],
        out_specs=pl.BlockSpec((1, N, O), lambda b: (b, 0, 0)),
        out_shape=jax.ShapeDtypeStruct((B, N, O), jnp.float32),
    )


def _edge_conv(xp, W, gam, bet, RB=512):
    B, N, C = xp.shape
    BN = B * N
    O, twoc = W.shape
    craw = twoc // 2
    wd = jnp.zeros((O, C), jnp.float32).at[:, :craw].set(W[:, :craw])
    wx = jnp.zeros((O, C), jnp.float32).at[:, :craw].set(W[:, craw:])
    idxt = _make_knn(B, N, C)(xp)
    g = _make_gather(BN, N, C)(xp.reshape(BN, C), idxt.reshape(B * KNN, N))
    m, sums = _make_conv_reduce(B, N, C, O, RB)(
        xp, g.reshape(KNN, B, N, C), wd, wx)
    return _make_norm(B, N, O)(m, sums, gam.reshape(1, O), bet.reshape(1, O))


def kernel(x, W1, g1, b1, W2, g2, b2, W3, g3, b3, W4, g4, b4):
    B, N, C0 = x.shape
    xp = jnp.pad(x, ((0, 0), (0, 0), (0, 16 - C0)))
    outs = []
    for W, gam, bet in ((W1, g1, b1), (W2, g2, b2), (W3, g3, b3), (W4, g4, b4)):
        xp = _edge_conv(xp, W, gam, bet)
        outs.append(xp)
    return jnp.concatenate(outs, axis=-1)
